# baseline (speedup 1.0000x reference)
import functools

import jax
import jax.numpy as jnp
from jax import lax
from jax.experimental import pallas as pl
from jax.experimental.pallas import tpu as pltpu
from jax.experimental.pallas import tpu_sc as plsc

N = 10000
N_PAD = 10240
NW = 32
CHUNK = 128
STEPS = 80
E_PAD = NW * STEPS * CHUNK
RPS = N_PAD // 16
SSTEPS = 2 * STEPS
HALF = 64

_MESH = dict(core_axis_name="c", subcore_axis_name="s")


def _make_deg_kernel():
  @functools.partial(
      pl.kernel,
      out_type=jax.ShapeDtypeStruct((2, N_PAD), jnp.float32),
      mesh=plsc.VectorSubcoreMesh(**_MESH),
      scratch_types=[
          pltpu.VMEM((STEPS, CHUNK), jnp.int32),
          pltpu.VMEM((CHUNK,), jnp.float32),
          pltpu.VMEM_SHARED((N_PAD,), jnp.float32),
      ],
  )
  def deg_kernel(dst_hbm, zeros_hbm, out_hbm, didx_v, ones_v, acc_sh):
    c = lax.axis_index("c")
    s = lax.axis_index("s")
    w = s * 2 + c
    pltpu.sync_copy(dst_hbm.at[w], didx_v)
    for i in range(CHUNK // 16):
      ones_v[pl.ds(i * 16, 16)] = jnp.ones((16,), jnp.float32)
    pltpu.sync_copy(zeros_hbm.at[pl.ds(s * RPS, RPS)],
                    acc_sh.at[pl.ds(s * RPS, RPS)])
    plsc.subcore_barrier()

    def step(j, carry):
      pltpu.sync_copy(ones_v, acc_sh.at[didx_v.at[j]], add=True)
      return carry

    lax.fori_loop(0, STEPS, step, 0)
    plsc.subcore_barrier()
    pltpu.sync_copy(acc_sh.at[pl.ds(s * RPS, RPS)],
                    out_hbm.at[c, pl.ds(s * RPS, RPS)])

  return deg_kernel


def _make_scatter_kernel():
  @functools.partial(
      pl.kernel,
      out_type=jax.ShapeDtypeStruct((2, N_PAD, HALF), jnp.float32),
      mesh=plsc.VectorSubcoreMesh(**_MESH),
      scratch_types=[
          pltpu.VMEM((SSTEPS, CHUNK), jnp.int32),
          pltpu.VMEM((SSTEPS, CHUNK), jnp.int32),
          pltpu.VMEM((2, CHUNK, HALF), jnp.float32),
          pltpu.VMEM_SHARED((N_PAD, HALF), jnp.float32),
          pltpu.SemaphoreType.DMA,
          pltpu.SemaphoreType.DMA,
      ],
      compiler_params=pltpu.CompilerParams(use_tc_tiling_on_sc=False),
  )
  def scat_kernel(src_hbm, dst_hbm, y_hbm, zeros_hbm, out_hbm,
                  sidx_v, didx_v, rows_v, acc_sh, sg0, sg1):
    c = lax.axis_index("c")
    s = lax.axis_index("s")
    yv = y_hbm.at[c]
    pltpu.sync_copy(src_hbm.at[s], sidx_v)
    pltpu.sync_copy(dst_hbm.at[s], didx_v)
    pltpu.sync_copy(zeros_hbm.at[pl.ds(s * RPS, RPS)],
                    acc_sh.at[pl.ds(s * RPS, RPS)])
    plsc.subcore_barrier()

    pltpu.async_copy(yv.at[sidx_v.at[0]], rows_v.at[0], sg0)

    def pair(i, carry):
      j0 = 2 * i
      pltpu.make_async_copy(yv.at[sidx_v.at[j0]], rows_v.at[0], sg0).wait()
      pltpu.async_copy(yv.at[sidx_v.at[j0 + 1]], rows_v.at[1], sg1)
      pltpu.sync_copy(rows_v.at[0], acc_sh.at[didx_v.at[j0]], add=True)
      pltpu.make_async_copy(yv.at[sidx_v.at[j0 + 1]], rows_v.at[1],
                            sg1).wait()

      @pl.when(i < SSTEPS // 2 - 1)
      def _():
        pltpu.async_copy(yv.at[sidx_v.at[j0 + 2]], rows_v.at[0], sg0)

      pltpu.sync_copy(rows_v.at[1], acc_sh.at[didx_v.at[j0 + 1]], add=True)
      return carry

    lax.fori_loop(0, SSTEPS // 2, pair, 0)
    plsc.subcore_barrier()
    pltpu.sync_copy(acc_sh.at[pl.ds(s * RPS, RPS)],
                    out_hbm.at[c, pl.ds(s * RPS, RPS)])

  return scat_kernel


_DEG = _make_deg_kernel()
_SCATTER = _make_scatter_kernel()


def _dinv_body(da_ref, db_ref, o_ref):
  o_ref[...] = lax.rsqrt(da_ref[...] + db_ref[...] + 1.0)


def _tc_dinv(dega, degb):
  return pl.pallas_call(
      _dinv_body,
      out_shape=jax.ShapeDtypeStruct((N_PAD // 128, 128), jnp.float32),
  )(dega.reshape(N_PAD // 128, 128), degb.reshape(N_PAD // 128, 128))


def _mm_body(dinv_ref, h_ref, w_ref, y_ref):
  y = dinv_ref[...] * jnp.dot(
      h_ref[...], w_ref[...], preferred_element_type=jnp.float32)
  y_ref[0] = y[:, :HALF]
  y_ref[1] = y[:, HALF:]


def _tc_matmul(h, W, dinv_col):
  n, din = h.shape
  blk = 1024
  return pl.pallas_call(
      _mm_body,
      grid=(n // blk,),
      in_specs=[
          pl.BlockSpec((blk, 1), lambda i: (i, 0)),
          pl.BlockSpec((blk, din), lambda i: (i, 0)),
          pl.BlockSpec((din, 128), lambda i: (0, 0)),
      ],
      out_specs=pl.BlockSpec((2, blk, HALF), lambda i: (0, i, 0)),
      out_shape=jax.ShapeDtypeStruct((2, n, HALF), jnp.float32),
  )(dinv_col, h, W)


def _comb_body(dinv_ref, acc_ref, y_ref, b_ref, o_ref):
  dinv = dinv_ref[...]
  o_ref[:, :HALF] = jnp.maximum(
      dinv * (acc_ref[0] + y_ref[0]) + b_ref[:, :HALF], 0.0)
  o_ref[:, HALF:] = jnp.maximum(
      dinv * (acc_ref[1] + y_ref[1]) + b_ref[:, HALF:], 0.0)


def _tc_combine(accs, y, dinv_col, b):
  n = y.shape[1]
  blk = 1024
  return pl.pallas_call(
      _comb_body,
      grid=(n // blk,),
      in_specs=[
          pl.BlockSpec((blk, 1), lambda i: (i, 0)),
          pl.BlockSpec((2, blk, HALF), lambda i: (0, i, 0)),
          pl.BlockSpec((2, blk, HALF), lambda i: (0, i, 0)),
          pl.BlockSpec((1, 128), lambda i: (0, 0)),
      ],
      out_specs=pl.BlockSpec((blk, 128), lambda i: (i, 0)),
      out_shape=jax.ShapeDtypeStruct((n, 128), jnp.float32),
  )(dinv_col, accs, y, b)


def kernel(x, edge_index, enc_W0, enc_b0, enc_W1, enc_b1,
           dec_W0, dec_b0, dec_W1, dec_b1):
  src = edge_index[0]
  dst = edge_index[1]
  pad = E_PAD - src.shape[0]
  src_p = jnp.concatenate([src, jnp.zeros((pad,), jnp.int32)])
  dst_p = jnp.concatenate([dst, jnp.full((pad,), N, jnp.int32)])
  src16 = src_p.reshape(16, SSTEPS, CHUNK)
  dst16 = dst_p.reshape(16, SSTEPS, CHUNK)
  dst32 = dst_p.reshape(NW, STEPS, CHUNK)
  zeros1 = jnp.zeros((N_PAD,), jnp.float32)
  zeros2 = jnp.zeros((N_PAD, HALF), jnp.float32)

  deg = _DEG(dst32, zeros1)
  dinv_col = _tc_dinv(deg[0], deg[1]).reshape(N_PAD, 1)

  h = jnp.pad(x, ((0, N_PAD - N), (0, 0)))
  for W, b in ((enc_W0, enc_b0), (enc_W1, enc_b1),
               (dec_W0, dec_b0), (dec_W1, dec_b1)):
    Wp = jnp.pad(W, ((0, 128 - W.shape[0]), (0, 128 - W.shape[1])))
    bp = jnp.pad(b, (0, 128 - b.shape[0]))
    y = _tc_matmul(h, Wp, dinv_col)
    accs = _SCATTER(src16, dst16, y, zeros2)
    h = _tc_combine(accs, y, dinv_col, bp.reshape(1, 128))
  return h[:N]


# speedup vs baseline: 1.1189x; 1.1189x over previous
import functools

import jax
import jax.numpy as jnp
from jax import lax
from jax.experimental import pallas as pl
from jax.experimental.pallas import tpu as pltpu
from jax.experimental.pallas import tpu_sc as plsc

N = 10000
N_PAD = 10240
NW = 32
CHUNK = 128
STEPS = 80
E_PAD = NW * STEPS * CHUNK
RPS = N_PAD // 16
SSTEPS = 2 * STEPS
HALF = 64

_MESH = dict(core_axis_name="c", subcore_axis_name="s")


def _make_deg_kernel():
  @functools.partial(
      pl.kernel,
      out_type=jax.ShapeDtypeStruct((2, N_PAD), jnp.float32),
      mesh=plsc.VectorSubcoreMesh(**_MESH),
      scratch_types=[
          pltpu.VMEM((STEPS, CHUNK), jnp.int32),
          pltpu.VMEM((CHUNK,), jnp.float32),
          pltpu.VMEM_SHARED((N_PAD,), jnp.float32),
      ],
  )
  def deg_kernel(dst_hbm, zeros_hbm, out_hbm, didx_v, ones_v, acc_sh):
    c = lax.axis_index("c")
    s = lax.axis_index("s")
    w = s * 2 + c
    pltpu.sync_copy(dst_hbm.at[w], didx_v)
    for i in range(CHUNK // 16):
      ones_v[pl.ds(i * 16, 16)] = jnp.ones((16,), jnp.float32)
    pltpu.sync_copy(zeros_hbm.at[pl.ds(s * RPS, RPS)],
                    acc_sh.at[pl.ds(s * RPS, RPS)])
    plsc.subcore_barrier()

    def step(j, carry):
      pltpu.sync_copy(ones_v, acc_sh.at[didx_v.at[j]], add=True)
      return carry

    lax.fori_loop(0, STEPS, step, 0)
    plsc.subcore_barrier()
    pltpu.sync_copy(acc_sh.at[pl.ds(s * RPS, RPS)],
                    out_hbm.at[c, pl.ds(s * RPS, RPS)])

  return deg_kernel


def _make_scatter_kernel():
  @functools.partial(
      pl.kernel,
      out_type=jax.ShapeDtypeStruct((2, N_PAD, HALF), jnp.float32),
      mesh=plsc.VectorSubcoreMesh(**_MESH),
      scratch_types=[
          pltpu.VMEM((SSTEPS, CHUNK), jnp.int32),
          pltpu.VMEM((SSTEPS, CHUNK), jnp.int32),
          pltpu.VMEM((4, CHUNK, HALF), jnp.float32),
          pltpu.VMEM_SHARED((N_PAD, HALF), jnp.float32),
      ] + [pltpu.SemaphoreType.DMA] * 8,
      compiler_params=pltpu.CompilerParams(use_tc_tiling_on_sc=False),
  )
  def scat_kernel(src_hbm, dst_hbm, y_hbm, zeros_hbm, out_hbm,
                  sidx_v, didx_v, rows_v, acc_sh, *sems):
    sg = sems[:4]
    ss = sems[4:]
    c = lax.axis_index("c")
    s = lax.axis_index("s")
    yv = y_hbm.at[c]
    pltpu.sync_copy(src_hbm.at[s], sidx_v)
    pltpu.sync_copy(dst_hbm.at[s], didx_v)
    pltpu.sync_copy(zeros_hbm.at[pl.ds(s * RPS, RPS)],
                    acc_sh.at[pl.ds(s * RPS, RPS)])
    plsc.subcore_barrier()

    for b in range(4):
      pltpu.async_copy(yv.at[sidx_v.at[b]], rows_v.at[b], sg[b])

    def group(i, carry):
      base = 4 * i
      descs = []
      for b in range(4):
        j = base + b
        pltpu.make_async_copy(yv.at[sidx_v.at[j]], rows_v.at[b],
                              sg[b]).wait()
        descs.append(pltpu.async_copy(rows_v.at[b],
                                      acc_sh.at[didx_v.at[j]],
                                      ss[b], add=True))
      for b in range(4):
        descs[b].wait()

        @pl.when(i < SSTEPS // 4 - 1)
        def _():
          pltpu.async_copy(yv.at[sidx_v.at[base + 4 + b]], rows_v.at[b],
                           sg[b])
      return carry

    lax.fori_loop(0, SSTEPS // 4, group, 0)
    plsc.subcore_barrier()
    pltpu.sync_copy(acc_sh.at[pl.ds(s * RPS, RPS)],
                    out_hbm.at[c, pl.ds(s * RPS, RPS)])

  return scat_kernel


_DEG = _make_deg_kernel()
_SCATTER = _make_scatter_kernel()


def _dinv_body(da_ref, db_ref, o_ref):
  o_ref[...] = lax.rsqrt(da_ref[...] + db_ref[...] + 1.0)


def _tc_dinv(dega, degb):
  return pl.pallas_call(
      _dinv_body,
      out_shape=jax.ShapeDtypeStruct((N_PAD // 128, 128), jnp.float32),
  )(dega.reshape(N_PAD // 128, 128), degb.reshape(N_PAD // 128, 128))


def _mm_body(dinv_ref, h_ref, w_ref, y_ref):
  y = dinv_ref[...] * jnp.dot(
      h_ref[...], w_ref[...], preferred_element_type=jnp.float32)
  y_ref[0] = y[:, :HALF]
  y_ref[1] = y[:, HALF:]


def _tc_matmul(h, W, dinv_col):
  n, din = h.shape
  blk = 1024
  return pl.pallas_call(
      _mm_body,
      grid=(n // blk,),
      in_specs=[
          pl.BlockSpec((blk, 1), lambda i: (i, 0)),
          pl.BlockSpec((blk, din), lambda i: (i, 0)),
          pl.BlockSpec((din, 128), lambda i: (0, 0)),
      ],
      out_specs=pl.BlockSpec((2, blk, HALF), lambda i: (0, i, 0)),
      out_shape=jax.ShapeDtypeStruct((2, n, HALF), jnp.float32),
  )(dinv_col, h, W)


def _comb_body(dinv_ref, acc_ref, y_ref, b_ref, o_ref):
  dinv = dinv_ref[...]
  o_ref[:, :HALF] = jnp.maximum(
      dinv * (acc_ref[0] + y_ref[0]) + b_ref[:, :HALF], 0.0)
  o_ref[:, HALF:] = jnp.maximum(
      dinv * (acc_ref[1] + y_ref[1]) + b_ref[:, HALF:], 0.0)


def _tc_combine(accs, y, dinv_col, b):
  n = y.shape[1]
  blk = 1024
  return pl.pallas_call(
      _comb_body,
      grid=(n // blk,),
      in_specs=[
          pl.BlockSpec((blk, 1), lambda i: (i, 0)),
          pl.BlockSpec((2, blk, HALF), lambda i: (0, i, 0)),
          pl.BlockSpec((2, blk, HALF), lambda i: (0, i, 0)),
          pl.BlockSpec((1, 128), lambda i: (0, 0)),
      ],
      out_specs=pl.BlockSpec((blk, 128), lambda i: (i, 0)),
      out_shape=jax.ShapeDtypeStruct((n, 128), jnp.float32),
  )(dinv_col, accs, y, b)


def kernel(x, edge_index, enc_W0, enc_b0, enc_W1, enc_b1,
           dec_W0, dec_b0, dec_W1, dec_b1):
  src = edge_index[0]
  dst = edge_index[1]
  pad = E_PAD - src.shape[0]
  src_p = jnp.concatenate([src, jnp.zeros((pad,), jnp.int32)])
  dst_p = jnp.concatenate([dst, jnp.full((pad,), N, jnp.int32)])
  src16 = src_p.reshape(16, SSTEPS, CHUNK)
  dst16 = dst_p.reshape(16, SSTEPS, CHUNK)
  dst32 = dst_p.reshape(NW, STEPS, CHUNK)
  zeros1 = jnp.zeros((N_PAD,), jnp.float32)
  zeros2 = jnp.zeros((N_PAD, HALF), jnp.float32)

  deg = _DEG(dst32, zeros1)
  dinv_col = _tc_dinv(deg[0], deg[1]).reshape(N_PAD, 1)

  h = jnp.pad(x, ((0, N_PAD - N), (0, 0)))
  for W, b in ((enc_W0, enc_b0), (enc_W1, enc_b1),
               (dec_W0, dec_b0), (dec_W1, dec_b1)):
    Wp = jnp.pad(W, ((0, 128 - W.shape[0]), (0, 128 - W.shape[1])))
    bp = jnp.pad(b, (0, 128 - b.shape[0]))
    y = _tc_matmul(h, Wp, dinv_col)
    accs = _SCATTER(src16, dst16, y, zeros2)
    h = _tc_combine(accs, y, dinv_col, bp.reshape(1, 128))
  return h[:N]


# speedup vs baseline: 1.2367x; 1.1053x over previous
import functools

import jax
import jax.numpy as jnp
from jax import lax
from jax.experimental import pallas as pl
from jax.experimental.pallas import tpu as pltpu
from jax.experimental.pallas import tpu_sc as plsc

N = 10000
N_PAD = 10240
NW = 32
CHUNK = 128
STEPS = 80
E_PAD = NW * STEPS * CHUNK
RPS = N_PAD // 16
SSTEPS = 2 * STEPS
HALF = 64

_MESH = dict(core_axis_name="c", subcore_axis_name="s")


def _make_deg_kernel():
  @functools.partial(
      pl.kernel,
      out_type=jax.ShapeDtypeStruct((2, N_PAD), jnp.float32),
      mesh=plsc.VectorSubcoreMesh(**_MESH),
      scratch_types=[
          pltpu.VMEM((STEPS, CHUNK), jnp.int32),
          pltpu.VMEM((CHUNK,), jnp.float32),
          pltpu.VMEM_SHARED((N_PAD,), jnp.float32),
      ],
  )
  def deg_kernel(dst_hbm, zeros_hbm, out_hbm, didx_v, ones_v, acc_sh):
    c = lax.axis_index("c")
    s = lax.axis_index("s")
    w = s * 2 + c
    pltpu.sync_copy(dst_hbm.at[w], didx_v)
    for i in range(CHUNK // 16):
      ones_v[pl.ds(i * 16, 16)] = jnp.ones((16,), jnp.float32)
    pltpu.sync_copy(zeros_hbm.at[pl.ds(s * RPS, RPS)],
                    acc_sh.at[pl.ds(s * RPS, RPS)])
    plsc.subcore_barrier()

    def step(j, carry):
      pltpu.sync_copy(ones_v, acc_sh.at[didx_v.at[j]], add=True)
      return carry

    lax.fori_loop(0, STEPS, step, 0)
    plsc.subcore_barrier()
    pltpu.sync_copy(acc_sh.at[pl.ds(s * RPS, RPS)],
                    out_hbm.at[c, pl.ds(s * RPS, RPS)])

  return deg_kernel


def _make_scatter_kernel():
  @functools.partial(
      pl.kernel,
      out_type=jax.ShapeDtypeStruct((2, N_PAD, HALF), jnp.float32),
      mesh=plsc.VectorSubcoreMesh(**_MESH),
      scratch_types=[
          pltpu.VMEM((SSTEPS, CHUNK), jnp.int32),
          pltpu.VMEM((SSTEPS, CHUNK), jnp.int32),
          pltpu.VMEM((4, CHUNK, HALF), jnp.float32),
          pltpu.VMEM_SHARED((N_PAD, HALF), jnp.float32),
      ] + [pltpu.SemaphoreType.DMA] * 8,
      compiler_params=pltpu.CompilerParams(use_tc_tiling_on_sc=False),
  )
  def scat_kernel(src_hbm, dst_hbm, y_hbm, zeros_hbm, out_hbm,
                  sidx_v, didx_v, rows_v, acc_sh, *sems):
    sg = sems[:4]
    ss = sems[4:]
    c = lax.axis_index("c")
    s = lax.axis_index("s")
    yv = y_hbm.at[c]
    pltpu.sync_copy(src_hbm.at[s], sidx_v)
    pltpu.sync_copy(dst_hbm.at[s], didx_v)
    pltpu.sync_copy(zeros_hbm, rows_v.at[0])
    for r in range(RPS // CHUNK):
      pltpu.sync_copy(rows_v.at[0],
                      acc_sh.at[pl.ds(s * RPS + r * CHUNK, CHUNK)])
    plsc.subcore_barrier()

    for b in range(4):
      pltpu.async_copy(yv.at[sidx_v.at[b]], rows_v.at[b], sg[b])

    def group(i, carry):
      base = 4 * i
      descs = []
      for b in range(4):
        j = base + b
        pltpu.make_async_copy(yv.at[sidx_v.at[j]], rows_v.at[b],
                              sg[b]).wait()
        descs.append(pltpu.async_copy(rows_v.at[b],
                                      acc_sh.at[didx_v.at[j]],
                                      ss[b], add=True))
      for b in range(4):
        descs[b].wait()

        @pl.when(i < SSTEPS // 4 - 1)
        def _():
          pltpu.async_copy(yv.at[sidx_v.at[base + 4 + b]], rows_v.at[b],
                           sg[b])
      return carry

    lax.fori_loop(0, SSTEPS // 4, group, 0)
    plsc.subcore_barrier()
    pltpu.sync_copy(acc_sh.at[pl.ds(s * RPS, RPS)],
                    out_hbm.at[c, pl.ds(s * RPS, RPS)])

  return scat_kernel


_DEG = _make_deg_kernel()
_SCATTER = _make_scatter_kernel()


_BLK = 1024


def _mm0_body(da_ref, db_ref, h_ref, w_ref, y_ref, dinv_ref):
  dinv = lax.rsqrt(da_ref[...] + db_ref[...] + 1.0)
  dinv_ref[...] = dinv
  y = dinv * jnp.dot(h_ref[...], w_ref[...],
                     preferred_element_type=jnp.float32)
  y_ref[0] = y[:, :HALF]
  y_ref[1] = y[:, HALF:]


def _tc_mm0(dega, degb, h, W):
  n = h.shape[0]
  return pl.pallas_call(
      _mm0_body,
      grid=(n // _BLK,),
      in_specs=[
          pl.BlockSpec((_BLK, 1), lambda i: (i, 0)),
          pl.BlockSpec((_BLK, 1), lambda i: (i, 0)),
          pl.BlockSpec((_BLK, 128), lambda i: (i, 0)),
          pl.BlockSpec((128, 128), lambda i: (0, 0)),
      ],
      out_specs=[
          pl.BlockSpec((2, _BLK, HALF), lambda i: (0, i, 0)),
          pl.BlockSpec((_BLK, 1), lambda i: (i, 0)),
      ],
      out_shape=[
          jax.ShapeDtypeStruct((2, n, HALF), jnp.float32),
          jax.ShapeDtypeStruct((n, 1), jnp.float32),
      ],
  )(dega, degb, h, W)


def _bnd_body(dinv_ref, acc_ref, y_ref, b_ref, w_ref, y2_ref):
  dinv = dinv_ref[...]
  h = jnp.concatenate(
      [jnp.maximum(dinv * (acc_ref[0] + y_ref[0]) + b_ref[:, :HALF], 0.0),
       jnp.maximum(dinv * (acc_ref[1] + y_ref[1]) + b_ref[:, HALF:], 0.0)],
      axis=1)
  y2 = dinv * jnp.dot(h, w_ref[...], preferred_element_type=jnp.float32)
  y2_ref[0] = y2[:, :HALF]
  y2_ref[1] = y2[:, HALF:]


def _tc_boundary(accs, y, dinv_col, b, Wn):
  n = y.shape[1]
  return pl.pallas_call(
      _bnd_body,
      grid=(n // _BLK,),
      in_specs=[
          pl.BlockSpec((_BLK, 1), lambda i: (i, 0)),
          pl.BlockSpec((2, _BLK, HALF), lambda i: (0, i, 0)),
          pl.BlockSpec((2, _BLK, HALF), lambda i: (0, i, 0)),
          pl.BlockSpec((1, 128), lambda i: (0, 0)),
          pl.BlockSpec((128, 128), lambda i: (0, 0)),
      ],
      out_specs=pl.BlockSpec((2, _BLK, HALF), lambda i: (0, i, 0)),
      out_shape=jax.ShapeDtypeStruct((2, n, HALF), jnp.float32),
  )(dinv_col, accs, y, b, Wn)


def _fin_body(dinv_ref, acc_ref, y_ref, b_ref, o_ref):
  dinv = dinv_ref[...]
  o_ref[:, :HALF] = jnp.maximum(
      dinv * (acc_ref[0] + y_ref[0]) + b_ref[:, :HALF], 0.0)
  o_ref[:, HALF:] = jnp.maximum(
      dinv * (acc_ref[1] + y_ref[1]) + b_ref[:, HALF:], 0.0)


def _tc_final(accs, y, dinv_col, b):
  n = y.shape[1]
  return pl.pallas_call(
      _fin_body,
      grid=(n // _BLK,),
      in_specs=[
          pl.BlockSpec((_BLK, 1), lambda i: (i, 0)),
          pl.BlockSpec((2, _BLK, HALF), lambda i: (0, i, 0)),
          pl.BlockSpec((2, _BLK, HALF), lambda i: (0, i, 0)),
          pl.BlockSpec((1, 128), lambda i: (0, 0)),
      ],
      out_specs=pl.BlockSpec((_BLK, 128), lambda i: (i, 0)),
      out_shape=jax.ShapeDtypeStruct((n, 128), jnp.float32),
  )(dinv_col, accs, y, b)


def kernel(x, edge_index, enc_W0, enc_b0, enc_W1, enc_b1,
           dec_W0, dec_b0, dec_W1, dec_b1):
  src = edge_index[0]
  dst = edge_index[1]
  pad = E_PAD - src.shape[0]
  src_p = jnp.concatenate([src, jnp.zeros((pad,), jnp.int32)])
  dst_p = jnp.concatenate([dst, jnp.full((pad,), N, jnp.int32)])
  src16 = src_p.reshape(16, SSTEPS, CHUNK)
  dst16 = dst_p.reshape(16, SSTEPS, CHUNK)
  dst32 = dst_p.reshape(NW, STEPS, CHUNK)
  zeros1 = jnp.zeros((N_PAD,), jnp.float32)
  zeros2 = jnp.zeros((CHUNK, HALF), jnp.float32)

  Ws, bs = [], []
  for W, b in ((enc_W0, enc_b0), (enc_W1, enc_b1),
               (dec_W0, dec_b0), (dec_W1, dec_b1)):
    Ws.append(jnp.pad(W, ((0, 128 - W.shape[0]), (0, 128 - W.shape[1]))))
    bs.append(jnp.pad(b, (0, 128 - b.shape[0])).reshape(1, 128))

  deg = _DEG(dst32, zeros1)
  h0 = jnp.pad(x, ((0, N_PAD - N), (0, 0)))
  y, dinv_col = _tc_mm0(deg[0].reshape(N_PAD, 1), deg[1].reshape(N_PAD, 1),
                        h0, Ws[0])
  for l in range(3):
    accs = _SCATTER(src16, dst16, y, zeros2)
    y = _tc_boundary(accs, y, dinv_col, bs[l], Ws[l + 1])
  accs = _SCATTER(src16, dst16, y, zeros2)
  h = _tc_final(accs, y, dinv_col, bs[3])
  return h[:N]
